# baseline (speedup 1.0000x reference)
import functools

import jax
import jax.numpy as jnp
from jax import lax
from jax.experimental import pallas as pl
from jax.experimental.pallas import tpu as pltpu
from jax.experimental.pallas import tpu_sc as plsc

NC = 2
NS = 16
NW = NC * NS
L = 16

NIMG = 4096
CELLS = 81
EMB = 64
NCOL = 16
SPAN = NIMG // NW

_mesh = plsc.VectorSubcoreMesh(core_axis_name="c", subcore_axis_name="s")


@functools.partial(
    pl.kernel,
    mesh=_mesh,
    out_type=jax.ShapeDtypeStruct((CELLS, NIMG, 2 * EMB), jnp.float32),
    scratch_types=[
        pltpu.VMEM((2, SPAN), jnp.int32),
        pltpu.VMEM((2, SPAN), jnp.int32),
        pltpu.VMEM((1, SPAN), jnp.int32),
        pltpu.VMEM((1, SPAN), jnp.int32),
        pltpu.VMEM((SPAN, 2 * EMB), jnp.float32),
        pltpu.VMEM((SPAN, 2 * EMB), jnp.float32),
        pltpu.SemaphoreType.DMA,
        pltpu.SemaphoreType.DMA,
        pltpu.SemaphoreType.DMA,
    ],
)
def _encode(img_hbm, comb_hbm, out_hbm,
            raw_a, raw_b, idx_a, idx_b, stage_a, stage_b,
            sem_ga, sem_gb, sem_w):
    wid = lax.axis_index("s") * NC + lax.axis_index("c")

    def prep_idx(s, raw, idx):
        base = (s * NW + wid) * SPAN
        pltpu.sync_copy(img_hbm.at[pl.ds(base, SPAN)], raw.at[0])
        pltpu.sync_copy(img_hbm.at[pl.ds(CELLS * NIMG + base, SPAN)],
                        raw.at[1])
        for k in range(SPAN // L):
            e = raw[0, pl.ds(L * k, L)]
            c = raw[1, pl.ds(L * k, L)]
            idx[0, pl.ds(L * k, L)] = e * NCOL + c

    def fire_gather(idx, stage, sem):
        pltpu.async_copy(comb_hbm.at[idx.at[0]], stage, sem)

    def drain_gather(stage, sem):
        pltpu.make_async_copy(comb_hbm.at[pl.ds(0, SPAN)], stage, sem).wait()

    def fire_write(s, stage):
        pltpu.async_copy(stage, out_hbm.at[s, pl.ds(SPAN * wid, SPAN)], sem_w)

    def drain_write(stage):
        pltpu.make_async_copy(stage,
                              out_hbm.at[0, pl.ds(0, SPAN)], sem_w).wait()

    def step(s, raw_c, idx_c, stage_c, sem_gc, raw_n, idx_n, stage_n, sem_gn):
        @pl.when(s + 1 < CELLS)
        def _():
            prep_idx(s + 1, raw_n, idx_n)
        @pl.when(s >= 1)
        def _():
            drain_write(stage_n)
        @pl.when(s + 1 < CELLS)
        def _():
            fire_gather(idx_n, stage_n, sem_gn)
        drain_gather(stage_c, sem_gc)
        fire_write(s, stage_c)

    prep_idx(0, raw_a, idx_a)
    fire_gather(idx_a, stage_a, sem_ga)

    def body(s, carry):
        @pl.when(s % 2 == 0)
        def _():
            step(s, raw_a, idx_a, stage_a, sem_ga,
                 raw_b, idx_b, stage_b, sem_gb)
        @pl.when(s % 2 == 1)
        def _():
            step(s, raw_b, idx_b, stage_b, sem_gb,
                 raw_a, idx_a, stage_a, sem_ga)
        return carry

    lax.fori_loop(0, CELLS, body, 0)
    drain_write(stage_a)


def kernel(img, entity_table, color_table):
    comb = jnp.concatenate(
        [jnp.repeat(entity_table[:NCOL], NCOL, axis=0),
         jnp.tile(color_table, (NCOL, 1))], axis=1)
    img_t = jnp.transpose(img, (3, 1, 2, 0)).reshape(2 * CELLS * NIMG)
    out = _encode(img_t, comb)
    return out.reshape(9, 9, NIMG, 2 * EMB).transpose(2, 0, 1, 3)


# speedup vs baseline: 2.6876x; 2.6876x over previous
import functools

import jax
import jax.numpy as jnp
from jax import lax
from jax.experimental import pallas as pl
from jax.experimental.pallas import tpu as pltpu
from jax.experimental.pallas import tpu_sc as plsc

NC = 2
NS = 16
NW = NC * NS
L = 16

NIMG = 4096
CELLS = 81
EMB = 64
NCOL = 16
SPAN = NIMG // NW

_mesh = plsc.VectorSubcoreMesh(core_axis_name="c", subcore_axis_name="s")


@functools.partial(
    pl.kernel,
    mesh=_mesh,
    out_type=jax.ShapeDtypeStruct((CELLS, NIMG, 2 * EMB), jnp.float32),
    scratch_types=[
        pltpu.VMEM((2, SPAN), jnp.int32),
        pltpu.VMEM((2, SPAN), jnp.int32),
        pltpu.VMEM((1, SPAN), jnp.int32),
        pltpu.VMEM((1, SPAN), jnp.int32),
        pltpu.VMEM((SPAN, 2 * EMB), jnp.float32),
        pltpu.VMEM((SPAN, 2 * EMB), jnp.float32),
        pltpu.VMEM_SHARED((NCOL * NCOL, 2 * EMB), jnp.float32),
        pltpu.SemaphoreType.DMA,
        pltpu.SemaphoreType.DMA,
        pltpu.SemaphoreType.DMA,
    ],
)
def _encode(img_hbm, comb_hbm, out_hbm,
            raw_a, raw_b, idx_a, idx_b, stage_a, stage_b, comb_sh,
            sem_ga, sem_gb, sem_w):
    wid = lax.axis_index("s") * NC + lax.axis_index("c")

    @pl.when(lax.axis_index("s") == 0)
    def _():
        pltpu.sync_copy(comb_hbm, comb_sh)
    plsc.subcore_barrier()

    def prep_idx(s, raw, idx):
        base = (s * NW + wid) * SPAN
        pltpu.sync_copy(img_hbm.at[pl.ds(base, SPAN)], raw.at[0])
        pltpu.sync_copy(img_hbm.at[pl.ds(CELLS * NIMG + base, SPAN)],
                        raw.at[1])
        for k in range(SPAN // L):
            e = raw[0, pl.ds(L * k, L)]
            c = raw[1, pl.ds(L * k, L)]
            idx[0, pl.ds(L * k, L)] = e * NCOL + c

    def fire_gather(idx, stage, sem):
        pltpu.async_copy(comb_sh.at[idx.at[0]], stage, sem)

    def drain_gather(stage, sem):
        pltpu.make_async_copy(comb_sh.at[pl.ds(0, SPAN)], stage, sem).wait()

    def fire_write(s, stage):
        pltpu.async_copy(stage, out_hbm.at[s, pl.ds(SPAN * wid, SPAN)], sem_w)

    def drain_write(stage):
        pltpu.make_async_copy(stage,
                              out_hbm.at[0, pl.ds(0, SPAN)], sem_w).wait()

    def step(s, raw_c, idx_c, stage_c, sem_gc, raw_n, idx_n, stage_n, sem_gn):
        @pl.when(s + 1 < CELLS)
        def _():
            prep_idx(s + 1, raw_n, idx_n)
        @pl.when(s >= 1)
        def _():
            drain_write(stage_n)
        @pl.when(s + 1 < CELLS)
        def _():
            fire_gather(idx_n, stage_n, sem_gn)
        drain_gather(stage_c, sem_gc)
        fire_write(s, stage_c)

    prep_idx(0, raw_a, idx_a)
    fire_gather(idx_a, stage_a, sem_ga)

    def body(s, carry):
        @pl.when(s % 2 == 0)
        def _():
            step(s, raw_a, idx_a, stage_a, sem_ga,
                 raw_b, idx_b, stage_b, sem_gb)
        @pl.when(s % 2 == 1)
        def _():
            step(s, raw_b, idx_b, stage_b, sem_gb,
                 raw_a, idx_a, stage_a, sem_ga)
        return carry

    lax.fori_loop(0, CELLS, body, 0)
    drain_write(stage_a)


def kernel(img, entity_table, color_table):
    comb = jnp.concatenate(
        [jnp.repeat(entity_table[:NCOL], NCOL, axis=0),
         jnp.tile(color_table, (NCOL, 1))], axis=1)
    img_t = jnp.transpose(img, (3, 1, 2, 0)).reshape(2 * CELLS * NIMG)
    out = _encode(img_t, comb)
    return out.reshape(9, 9, NIMG, 2 * EMB).transpose(2, 0, 1, 3)


# speedup vs baseline: 3.6939x; 1.3744x over previous
import functools

import jax
import jax.numpy as jnp
from jax import lax
from jax.experimental import pallas as pl
from jax.experimental.pallas import tpu as pltpu
from jax.experimental.pallas import tpu_sc as plsc

NC = 2
NS = 16
NW = NC * NS
L = 16

NIMG = 4096
CELLS = 81
EMB = 64
NCOL = 16
SPAN = NIMG // NW

_mesh = plsc.VectorSubcoreMesh(core_axis_name="c", subcore_axis_name="s")


@functools.partial(
    pl.kernel,
    mesh=_mesh,
    out_type=jax.ShapeDtypeStruct((CELLS, NIMG, 2 * EMB), jnp.float32),
    scratch_types=[
        pltpu.VMEM((2, SPAN), jnp.int32),
        pltpu.VMEM((2, SPAN), jnp.int32),
        pltpu.VMEM((1, SPAN), jnp.int32),
        pltpu.VMEM((1, SPAN), jnp.int32),
        pltpu.VMEM((SPAN, 2 * EMB), jnp.float32),
        pltpu.VMEM((SPAN, 2 * EMB), jnp.float32),
        pltpu.VMEM_SHARED((NCOL * NCOL, 2 * EMB), jnp.float32),
        pltpu.SemaphoreType.DMA,
        pltpu.SemaphoreType.DMA,
        pltpu.SemaphoreType.DMA,
        pltpu.SemaphoreType.DMA,
        pltpu.SemaphoreType.DMA,
    ],
)
def _encode(img_hbm, comb_hbm, out_hbm,
            raw_a, raw_b, idx_a, idx_b, stage_a, stage_b, comb_sh,
            sem_ga, sem_gb, sem_w, sem_ia, sem_ib):
    wid = lax.axis_index("s") * NC + lax.axis_index("c")

    @pl.when(lax.axis_index("s") == 0)
    def _():
        pltpu.sync_copy(comb_hbm, comb_sh)
    plsc.subcore_barrier()

    def fire_ids(s, raw, sem):
        base = (s * NW + wid) * SPAN
        pltpu.async_copy(img_hbm.at[pl.ds(base, SPAN)], raw.at[0], sem)
        pltpu.async_copy(img_hbm.at[pl.ds(CELLS * NIMG + base, SPAN)],
                         raw.at[1], sem)

    def prep_idx(raw, idx, sem):
        pltpu.make_async_copy(img_hbm.at[pl.ds(0, SPAN)], raw.at[0],
                              sem).wait()
        pltpu.make_async_copy(img_hbm.at[pl.ds(0, SPAN)], raw.at[1],
                              sem).wait()
        for k in range(SPAN // L):
            e = raw[0, pl.ds(L * k, L)]
            c = raw[1, pl.ds(L * k, L)]
            idx[0, pl.ds(L * k, L)] = e * NCOL + c

    def fire_gather(idx, stage, sem):
        pltpu.async_copy(comb_sh.at[idx.at[0]], stage, sem)

    def drain_gather(stage, sem):
        pltpu.make_async_copy(comb_sh.at[pl.ds(0, SPAN)], stage, sem).wait()

    def fire_write(s, stage):
        pltpu.async_copy(stage, out_hbm.at[s, pl.ds(SPAN * wid, SPAN)], sem_w)

    def drain_write(stage):
        pltpu.make_async_copy(stage,
                              out_hbm.at[0, pl.ds(0, SPAN)], sem_w).wait()

    def step(s, raw_c, idx_c, stage_c, sem_gc, sem_ic,
             raw_n, idx_n, stage_n, sem_gn, sem_in):
        @pl.when(s + 2 < CELLS)
        def _():
            fire_ids(s + 2, raw_c, sem_ic)
        @pl.when(s + 1 < CELLS)
        def _():
            prep_idx(raw_n, idx_n, sem_in)
        @pl.when(s >= 1)
        def _():
            drain_write(stage_n)
        @pl.when(s + 1 < CELLS)
        def _():
            fire_gather(idx_n, stage_n, sem_gn)
        drain_gather(stage_c, sem_gc)
        fire_write(s, stage_c)

    fire_ids(0, raw_a, sem_ia)
    fire_ids(1, raw_b, sem_ib)
    prep_idx(raw_a, idx_a, sem_ia)
    fire_gather(idx_a, stage_a, sem_ga)

    def body(s, carry):
        @pl.when(s % 2 == 0)
        def _():
            step(s, raw_a, idx_a, stage_a, sem_ga, sem_ia,
                 raw_b, idx_b, stage_b, sem_gb, sem_ib)
        @pl.when(s % 2 == 1)
        def _():
            step(s, raw_b, idx_b, stage_b, sem_gb, sem_ib,
                 raw_a, idx_a, stage_a, sem_ga, sem_ia)
        return carry

    lax.fori_loop(0, CELLS, body, 0)
    drain_write(stage_a)


def kernel(img, entity_table, color_table):
    comb = jnp.concatenate(
        [jnp.repeat(entity_table[:NCOL], NCOL, axis=0),
         jnp.tile(color_table, (NCOL, 1))], axis=1)
    img_t = jnp.transpose(img, (3, 1, 2, 0)).reshape(2 * CELLS * NIMG)
    out = _encode(img_t, comb)
    return out.reshape(9, 9, NIMG, 2 * EMB).transpose(2, 0, 1, 3)
